# baseline (speedup 1.0000x reference)
import jax
import jax.numpy as jnp
from jax import lax
from jax.experimental import pallas as pl
from jax.experimental.pallas import tpu as pltpu
from jax.experimental.pallas import tpu_sc as plsc

N_NODES = 10000
N_EDGES = 320000
D_FEAT = 128

NUM_CORES = 2
NUM_SUBCORES = 16
D_HALF = D_FEAT // NUM_CORES

NBUF = 2
CHUNK = 128
CHUNKS_PER_TILE = NBUF * -(-N_EDGES // (NUM_SUBCORES * CHUNK * NBUF))
EDGES_PER_TILE = CHUNKS_PER_TILE * CHUNK
E_PAD = EDGES_PER_TILE * NUM_SUBCORES

ACC_ROWS = 10112
ROWS_PER_SUBCORE = ACC_ROWS // NUM_SUBCORES
DUMMY_ROW = N_NODES

DEG_W = 16


def _sc_body(emb_hbm, src_hbm, dst_hbm, ones_hbm, zeros_hbm, z16_hbm,
             part_hbm, deg_hbm,
             src_v, dst_v, rows0_v, rows1_v, rows2_v,
             ones_v, zb_v, z16_v, acc_sh, deg_sh, *sems):
  c = lax.axis_index("c")
  s = lax.axis_index("s")
  rows = [rows0_v, rows1_v, rows2_v]
  gsem = sems[:NBUF]
  ssem = sems[NBUF:2 * NBUF]
  dsem = sems[2 * NBUF:]

  pltpu.sync_copy(ones_hbm, ones_v)
  pltpu.sync_copy(zeros_hbm, zb_v)
  pltpu.sync_copy(z16_hbm, z16_v)

  base = s * ROWS_PER_SUBCORE
  for k in range(ROWS_PER_SUBCORE // CHUNK):
    pltpu.sync_copy(zb_v, acc_sh.at[pl.ds(base + k * CHUNK, CHUNK)])
    pltpu.sync_copy(z16_v, deg_sh.at[pl.ds(base + k * CHUNK, CHUNK)])
  rem = ROWS_PER_SUBCORE % CHUNK
  if rem:
    off = base + (ROWS_PER_SUBCORE // CHUNK) * CHUNK
    pltpu.sync_copy(zb_v.at[pl.ds(0, rem)], acc_sh.at[pl.ds(off, rem)])
    pltpu.sync_copy(z16_v.at[pl.ds(0, rem)], deg_sh.at[pl.ds(off, rem)])

  plsc.subcore_barrier()

  pltpu.sync_copy(src_hbm.at[s], src_v)
  pltpu.sync_copy(dst_hbm.at[s], dst_v)

  def gather(j, b):
    pltpu.async_copy(emb_hbm.at[c].at[src_v.at[j]], rows[b], gsem[b])

  def wait_gather(j, b):
    pltpu.make_async_copy(emb_hbm.at[c].at[src_v.at[j]], rows[b],
                          gsem[b]).wait()

  def scatter(j, b):
    pltpu.async_copy(rows[b], acc_sh.at[dst_v.at[j]], ssem[b], add=True)

    @pl.when(c == (b % 2))
    def _():
      pltpu.async_copy(ones_v, deg_sh.at[dst_v.at[j]], dsem[b], add=True)

  def wait_scatter(j, b):
    pltpu.make_async_copy(rows[b], acc_sh.at[dst_v.at[j]], ssem[b]).wait()

    @pl.when(c == (b % 2))
    def _():
      pltpu.make_async_copy(ones_v, deg_sh.at[dst_v.at[j]],
                            dsem[b]).wait()

  for b in range(NBUF):
    gather(b, b)

  @pl.loop(0, CHUNKS_PER_TILE // NBUF)
  def _(i):
    jbase = i * NBUF
    for b in range(NBUF):
      j = jbase + b
      wait_gather(j, b)
      scatter(j, b)
      pb = (b - 1) % NBUF
      pj = j - 1 + NBUF

      @pl.when((pj >= NBUF) & (pj < CHUNKS_PER_TILE))
      def _():
        wait_scatter(pj - NBUF, pb)
        gather(pj, pb)

  for b in range(NBUF):
    wait_scatter(CHUNKS_PER_TILE - NBUF + b, b)

  plsc.subcore_barrier()

  pltpu.sync_copy(acc_sh.at[pl.ds(base, ROWS_PER_SUBCORE)],
                  part_hbm.at[c, pl.ds(base, ROWS_PER_SUBCORE)])
  pltpu.sync_copy(deg_sh.at[pl.ds(base, ROWS_PER_SUBCORE)],
                  deg_hbm.at[c, pl.ds(base, ROWS_PER_SUBCORE)])


def _combine_body(p0_ref, p1_ref, d0_ref, d1_ref, o_ref):
  deg = jnp.maximum(d0_ref[:, 0:1] + d1_ref[:, 0:1], 1.0)
  o_ref[...] = jnp.concatenate([p0_ref[...], p1_ref[...]], axis=1) / deg


@jax.jit
def kernel(user_emb, edge_index):
  src = edge_index[0].astype(jnp.int32)
  dst = edge_index[1].astype(jnp.int32)
  pad = E_PAD - N_EDGES
  src = jnp.concatenate([src, jnp.zeros((pad,), jnp.int32)])
  dst = jnp.concatenate([dst, jnp.full((pad,), DUMMY_ROW, jnp.int32)])
  src_r = src.reshape(NUM_SUBCORES, CHUNKS_PER_TILE, CHUNK)
  dst_r = dst.reshape(NUM_SUBCORES, CHUNKS_PER_TILE, CHUNK)

  emb_t = user_emb.reshape(N_NODES, NUM_CORES, D_HALF).transpose(1, 0, 2)

  ones16 = jnp.ones((CHUNK, DEG_W), jnp.float32)
  zhalf = jnp.zeros((CHUNK, D_HALF), jnp.float32)
  z16 = jnp.zeros((CHUNK, DEG_W), jnp.float32)

  mesh = plsc.VectorSubcoreMesh(core_axis_name="c", subcore_axis_name="s")
  sc = pl.kernel(
      _sc_body,
      out_type=[
          jax.ShapeDtypeStruct((NUM_CORES, ACC_ROWS, D_HALF), jnp.float32),
          jax.ShapeDtypeStruct((NUM_CORES, ACC_ROWS, DEG_W), jnp.float32),
      ],
      mesh=mesh,
      compiler_params=pltpu.CompilerParams(use_tc_tiling_on_sc=False),
      scratch_types=[
          pltpu.VMEM((CHUNKS_PER_TILE, CHUNK), jnp.int32),
          pltpu.VMEM((CHUNKS_PER_TILE, CHUNK), jnp.int32),
          pltpu.VMEM((CHUNK, D_HALF), jnp.float32),
          pltpu.VMEM((CHUNK, D_HALF), jnp.float32),
          pltpu.VMEM((CHUNK, D_HALF), jnp.float32),
          pltpu.VMEM((CHUNK, DEG_W), jnp.float32),
          pltpu.VMEM((CHUNK, D_HALF), jnp.float32),
          pltpu.VMEM((CHUNK, DEG_W), jnp.float32),
          pltpu.VMEM_SHARED((ACC_ROWS, D_HALF), jnp.float32),
          pltpu.VMEM_SHARED((ACC_ROWS, DEG_W), jnp.float32),
      ] + [pltpu.SemaphoreType.DMA] * (3 * NBUF) + [
      ],
  )
  part, deg = sc(emb_t, src_r, dst_r, ones16, zhalf, z16)

  out = pl.pallas_call(
      _combine_body,
      out_shape=jax.ShapeDtypeStruct((N_NODES, D_FEAT), jnp.float32),
  )(part[0, :N_NODES], part[1, :N_NODES], deg[0, :N_NODES], deg[1, :N_NODES])
  return out


# speedup vs baseline: 1.1202x; 1.1202x over previous
import jax
import jax.numpy as jnp
from jax import lax
from jax.experimental import pallas as pl
from jax.experimental.pallas import tpu as pltpu
from jax.experimental.pallas import tpu_sc as plsc

N_NODES = 10000
N_EDGES = 320000
D_FEAT = 128

NUM_CORES = 2
NUM_SUBCORES = 16
D_HALF = D_FEAT // NUM_CORES

NBUF = 2
CHUNK = 128
CHUNKS_PER_TILE = NBUF * -(-N_EDGES // (NUM_SUBCORES * CHUNK * NBUF))
EDGES_PER_TILE = CHUNKS_PER_TILE * CHUNK
E_PAD = EDGES_PER_TILE * NUM_SUBCORES

ACC_ROWS = 10112
ROWS_PER_SUBCORE = ACC_ROWS // NUM_SUBCORES
DUMMY_ROW = N_NODES

DEG_W = 16


def _sc_body(emb_hbm, src_hbm, dst_hbm, ones_hbm, zeros_hbm, z16_hbm,
             part_hbm, deg_hbm,
             src_v, dst_v, rows0_v, rows1_v, rows2_v,
             ones_v, zb_v, z16_v, acc_sh, deg_sh, *sems):
  c = lax.axis_index("c")
  s = lax.axis_index("s")
  rows = [rows0_v, rows1_v, rows2_v]
  gsem = sems[:NBUF]
  ssem = sems[NBUF:2 * NBUF]
  dsem = sems[2 * NBUF:]

  pltpu.sync_copy(ones_hbm, ones_v)
  pltpu.sync_copy(zeros_hbm, zb_v)
  pltpu.sync_copy(z16_hbm, z16_v)

  base = s * ROWS_PER_SUBCORE
  for k in range(ROWS_PER_SUBCORE // CHUNK):
    pltpu.sync_copy(zb_v, acc_sh.at[pl.ds(base + k * CHUNK, CHUNK)])
    pltpu.sync_copy(z16_v, deg_sh.at[pl.ds(base + k * CHUNK, CHUNK)])
  rem = ROWS_PER_SUBCORE % CHUNK
  if rem:
    off = base + (ROWS_PER_SUBCORE // CHUNK) * CHUNK
    pltpu.sync_copy(zb_v.at[pl.ds(0, rem)], acc_sh.at[pl.ds(off, rem)])
    pltpu.sync_copy(z16_v.at[pl.ds(0, rem)], deg_sh.at[pl.ds(off, rem)])

  plsc.subcore_barrier()

  pltpu.sync_copy(src_hbm.at[s], src_v)
  pltpu.sync_copy(dst_hbm.at[s], dst_v)

  def gather(j, b):
    pltpu.async_copy(emb_hbm.at[c].at[src_v.at[j]], rows[b], gsem[b])

  def wait_gather(j, b):
    pltpu.make_async_copy(emb_hbm.at[c].at[src_v.at[j]], rows[b],
                          gsem[b]).wait()

  def consume(j, b):
    pltpu.sync_copy(rows[b], acc_sh.at[dst_v.at[j]], add=True)

    @pl.when(c == (b % 2))
    def _():
      pltpu.sync_copy(ones_v, deg_sh.at[dst_v.at[j]], add=True)

  gather(0, 0)

  @pl.loop(0, CHUNKS_PER_TILE // 2)
  def _(i):
    j0 = 2 * i
    gather(j0 + 1, 1)
    wait_gather(j0, 0)
    consume(j0, 0)

    @pl.when(j0 + 2 < CHUNKS_PER_TILE)
    def _():
      gather(j0 + 2, 0)

    wait_gather(j0 + 1, 1)
    consume(j0 + 1, 1)

  plsc.subcore_barrier()

  pltpu.sync_copy(acc_sh.at[pl.ds(base, ROWS_PER_SUBCORE)],
                  part_hbm.at[c, pl.ds(base, ROWS_PER_SUBCORE)])
  pltpu.sync_copy(deg_sh.at[pl.ds(base, ROWS_PER_SUBCORE)],
                  deg_hbm.at[c, pl.ds(base, ROWS_PER_SUBCORE)])


def _combine_body(p0_ref, p1_ref, d0_ref, d1_ref, o_ref):
  deg = jnp.maximum(d0_ref[:, 0:1] + d1_ref[:, 0:1], 1.0)
  o_ref[...] = jnp.concatenate([p0_ref[...], p1_ref[...]], axis=1) / deg


@jax.jit
def kernel(user_emb, edge_index):
  src = edge_index[0].astype(jnp.int32)
  dst = edge_index[1].astype(jnp.int32)
  pad = E_PAD - N_EDGES
  src = jnp.concatenate([src, jnp.zeros((pad,), jnp.int32)])
  dst = jnp.concatenate([dst, jnp.full((pad,), DUMMY_ROW, jnp.int32)])
  src_r = src.reshape(NUM_SUBCORES, CHUNKS_PER_TILE, CHUNK)
  dst_r = dst.reshape(NUM_SUBCORES, CHUNKS_PER_TILE, CHUNK)

  emb_t = user_emb.reshape(N_NODES, NUM_CORES, D_HALF).transpose(1, 0, 2)

  ones16 = jnp.ones((CHUNK, DEG_W), jnp.float32)
  zhalf = jnp.zeros((CHUNK, D_HALF), jnp.float32)
  z16 = jnp.zeros((CHUNK, DEG_W), jnp.float32)

  mesh = plsc.VectorSubcoreMesh(core_axis_name="c", subcore_axis_name="s")
  sc = pl.kernel(
      _sc_body,
      out_type=[
          jax.ShapeDtypeStruct((NUM_CORES, ACC_ROWS, D_HALF), jnp.float32),
          jax.ShapeDtypeStruct((NUM_CORES, ACC_ROWS, DEG_W), jnp.float32),
      ],
      mesh=mesh,
      compiler_params=pltpu.CompilerParams(use_tc_tiling_on_sc=False),
      scratch_types=[
          pltpu.VMEM((CHUNKS_PER_TILE, CHUNK), jnp.int32),
          pltpu.VMEM((CHUNKS_PER_TILE, CHUNK), jnp.int32),
          pltpu.VMEM((CHUNK, D_HALF), jnp.float32),
          pltpu.VMEM((CHUNK, D_HALF), jnp.float32),
          pltpu.VMEM((CHUNK, D_HALF), jnp.float32),
          pltpu.VMEM((CHUNK, DEG_W), jnp.float32),
          pltpu.VMEM((CHUNK, D_HALF), jnp.float32),
          pltpu.VMEM((CHUNK, DEG_W), jnp.float32),
          pltpu.VMEM_SHARED((ACC_ROWS, D_HALF), jnp.float32),
          pltpu.VMEM_SHARED((ACC_ROWS, DEG_W), jnp.float32),
      ] + [pltpu.SemaphoreType.DMA] * (3 * NBUF) + [
      ],
  )
  part, deg = sc(emb_t, src_r, dst_r, ones16, zhalf, z16)

  out = pl.pallas_call(
      _combine_body,
      out_shape=jax.ShapeDtypeStruct((N_NODES, D_FEAT), jnp.float32),
  )(part[0, :N_NODES], part[1, :N_NODES], deg[0, :N_NODES], deg[1, :N_NODES])
  return out


# speedup vs baseline: 1.1728x; 1.0469x over previous
import jax
import jax.numpy as jnp
from jax import lax
from jax.experimental import pallas as pl
from jax.experimental.pallas import tpu as pltpu
from jax.experimental.pallas import tpu_sc as plsc

N_NODES = 10000
N_EDGES = 320000
D_FEAT = 128

NUM_CORES = 2
NUM_SUBCORES = 16
NUM_LANES = 16
D_HALF = D_FEAT // NUM_CORES

CHUNK = 128
CHUNKS_PER_TILE = 2 * -(-N_EDGES // (NUM_SUBCORES * CHUNK * 2))
EDGES_PER_TILE = CHUNKS_PER_TILE * CHUNK
E_PAD = EDGES_PER_TILE * NUM_SUBCORES

ACC_ROWS = 10112
ROWS_PER_SUBCORE = ACC_ROWS // NUM_SUBCORES
DUMMY_ROW = N_NODES


def _sc_body(emb_hbm, src_hbm, dst_hbm, zeros_hbm,
             part_hbm, deg_hbm,
             src_v, dst_v, rows0_v, rows1_v, zb_v, hist_v, acc_sh,
             gsem0, gsem1):
  c = lax.axis_index("c")
  s = lax.axis_index("s")
  rows = [rows0_v, rows1_v]
  gsem = [gsem0, gsem1]

  pltpu.sync_copy(zeros_hbm, zb_v)

  @pl.loop(0, ACC_ROWS // NUM_LANES)
  def _(i):
    hist_v[pl.ds(i * NUM_LANES, NUM_LANES)] = jnp.zeros((NUM_LANES,),
                                                        jnp.float32)

  base = s * ROWS_PER_SUBCORE
  for k in range(ROWS_PER_SUBCORE // CHUNK):
    pltpu.sync_copy(zb_v, acc_sh.at[pl.ds(base + k * CHUNK, CHUNK)])
  rem = ROWS_PER_SUBCORE % CHUNK
  if rem:
    off = base + (ROWS_PER_SUBCORE // CHUNK) * CHUNK
    pltpu.sync_copy(zb_v.at[pl.ds(0, rem)], acc_sh.at[pl.ds(off, rem)])

  plsc.subcore_barrier()

  pltpu.sync_copy(src_hbm.at[s], src_v)
  pltpu.sync_copy(dst_hbm.at[s], dst_v)

  def gather(j, b):
    pltpu.async_copy(emb_hbm.at[c].at[src_v.at[j]], rows[b], gsem[b])

  def wait_gather(j, b):
    pltpu.make_async_copy(emb_hbm.at[c].at[src_v.at[j]], rows[b],
                          gsem[b]).wait()

  def consume(j, b):
    @pl.when(c == 0)
    def _():
      ones = jnp.ones((NUM_LANES,), jnp.float32)
      for k in range(CHUNK // NUM_LANES):
        idx = dst_v[j, pl.ds(k * NUM_LANES, NUM_LANES)]
        plsc.addupdate_scatter(hist_v, [idx], ones)

    pltpu.sync_copy(rows[b], acc_sh.at[dst_v.at[j]], add=True)

  gather(0, 0)

  @pl.loop(0, CHUNKS_PER_TILE // 2)
  def _(i):
    j0 = 2 * i
    gather(j0 + 1, 1)
    wait_gather(j0, 0)
    consume(j0, 0)

    @pl.when(j0 + 2 < CHUNKS_PER_TILE)
    def _():
      gather(j0 + 2, 0)

    wait_gather(j0 + 1, 1)
    consume(j0 + 1, 1)

  plsc.subcore_barrier()

  pltpu.sync_copy(acc_sh.at[pl.ds(base, ROWS_PER_SUBCORE)],
                  part_hbm.at[c, pl.ds(base, ROWS_PER_SUBCORE)])

  @pl.when(c == 0)
  def _():
    pltpu.sync_copy(hist_v, deg_hbm.at[s])


def _combine_body(p0_ref, p1_ref, d_ref, o_ref):
  deg = jnp.sum(d_ref[...], axis=0)[:N_NODES]
  deg = jnp.maximum(deg, 1.0).reshape(N_NODES, 1)
  o_ref[...] = jnp.concatenate([p0_ref[...], p1_ref[...]], axis=1) / deg


@jax.jit
def kernel(user_emb, edge_index):
  src = edge_index[0].astype(jnp.int32)
  dst = edge_index[1].astype(jnp.int32)
  pad = E_PAD - N_EDGES
  src = jnp.concatenate([src, jnp.zeros((pad,), jnp.int32)])
  dst = jnp.concatenate([dst, jnp.full((pad,), DUMMY_ROW, jnp.int32)])
  src_r = src.reshape(NUM_SUBCORES, CHUNKS_PER_TILE, CHUNK)
  dst_r = dst.reshape(NUM_SUBCORES, CHUNKS_PER_TILE, CHUNK)

  emb_t = user_emb.reshape(N_NODES, NUM_CORES, D_HALF).transpose(1, 0, 2)

  zhalf = jnp.zeros((CHUNK, D_HALF), jnp.float32)

  mesh = plsc.VectorSubcoreMesh(core_axis_name="c", subcore_axis_name="s")
  sc = pl.kernel(
      _sc_body,
      out_type=[
          jax.ShapeDtypeStruct((NUM_CORES, ACC_ROWS, D_HALF), jnp.float32),
          jax.ShapeDtypeStruct((NUM_SUBCORES, ACC_ROWS), jnp.float32),
      ],
      mesh=mesh,
      compiler_params=pltpu.CompilerParams(use_tc_tiling_on_sc=False,
                                           needs_layout_passes=False),
      scratch_types=[
          pltpu.VMEM((CHUNKS_PER_TILE, CHUNK), jnp.int32),
          pltpu.VMEM((CHUNKS_PER_TILE, CHUNK), jnp.int32),
          pltpu.VMEM((CHUNK, D_HALF), jnp.float32),
          pltpu.VMEM((CHUNK, D_HALF), jnp.float32),
          pltpu.VMEM((CHUNK, D_HALF), jnp.float32),
          pltpu.VMEM((ACC_ROWS,), jnp.float32),
          pltpu.VMEM_SHARED((ACC_ROWS, D_HALF), jnp.float32),
          pltpu.SemaphoreType.DMA,
          pltpu.SemaphoreType.DMA,
      ],
  )
  part, deg = sc(emb_t, src_r, dst_r, zhalf)

  out = pl.pallas_call(
      _combine_body,
      out_shape=jax.ShapeDtypeStruct((N_NODES, D_FEAT), jnp.float32),
  )(part[0, :N_NODES], part[1, :N_NODES], deg)
  return out


# speedup vs baseline: 1.2067x; 1.0289x over previous
import jax
import jax.numpy as jnp
from jax import lax
from jax.experimental import pallas as pl
from jax.experimental.pallas import tpu as pltpu
from jax.experimental.pallas import tpu_sc as plsc

N_NODES = 10000
N_EDGES = 320000
D_FEAT = 128

NUM_CORES = 2
NUM_SUBCORES = 16
NUM_LANES = 16
D_HALF = D_FEAT // NUM_CORES

CHUNK = 128
CHUNKS_PER_TILE = 2 * -(-N_EDGES // (NUM_SUBCORES * CHUNK * 2))
EDGES_PER_TILE = CHUNKS_PER_TILE * CHUNK
E_PAD = EDGES_PER_TILE * NUM_SUBCORES

ACC_ROWS = 10112
ROWS_PER_SUBCORE = ACC_ROWS // NUM_SUBCORES
DUMMY_ROW = N_NODES


def _sc_body(emb_hbm, src_hbm, dst_hbm, zeros_hbm,
             part_hbm, deg_hbm,
             src_v, dst_v, rows0_v, rows1_v, zb_v, hist_v, acc_sh,
             gsem0, gsem1):
  c = lax.axis_index("c")
  s = lax.axis_index("s")
  rows = [rows0_v, rows1_v]
  gsem = [gsem0, gsem1]

  pltpu.sync_copy(zeros_hbm, zb_v)

  @pl.loop(0, ACC_ROWS // NUM_LANES)
  def _(i):
    hist_v[pl.ds(i * NUM_LANES, NUM_LANES)] = jnp.zeros((NUM_LANES,),
                                                        jnp.float32)

  base = s * ROWS_PER_SUBCORE
  for k in range(ROWS_PER_SUBCORE // CHUNK):
    pltpu.sync_copy(zb_v, acc_sh.at[pl.ds(base + k * CHUNK, CHUNK)])
  rem = ROWS_PER_SUBCORE % CHUNK
  if rem:
    off = base + (ROWS_PER_SUBCORE // CHUNK) * CHUNK
    pltpu.sync_copy(zb_v.at[pl.ds(0, rem)], acc_sh.at[pl.ds(off, rem)])

  plsc.subcore_barrier()

  pltpu.sync_copy(src_hbm.at[s], src_v)
  pltpu.sync_copy(dst_hbm.at[s], dst_v)

  emb_half = emb_hbm.at[c]

  def gather(j, b):
    pltpu.async_copy(emb_half.at[src_v.at[j]], rows[b], gsem[b])

  def wait_gather(j, b):
    pltpu.make_async_copy(emb_half.at[src_v.at[j]], rows[b],
                          gsem[b]).wait()

  def consume(j, b):
    @pl.when(c == 0)
    def _():
      ones = jnp.ones((NUM_LANES,), jnp.float32)
      for k in range(CHUNK // NUM_LANES):
        idx = dst_v[j, pl.ds(k * NUM_LANES, NUM_LANES)]
        plsc.addupdate_scatter(hist_v, [idx], ones)

    pltpu.sync_copy(rows[b], acc_sh.at[dst_v.at[j]], add=True)

  gather(0, 0)

  @pl.loop(0, CHUNKS_PER_TILE // 2)
  def _(i):
    j0 = 2 * i
    gather(j0 + 1, 1)
    wait_gather(j0, 0)
    consume(j0, 0)

    @pl.when(j0 + 2 < CHUNKS_PER_TILE)
    def _():
      gather(j0 + 2, 0)

    wait_gather(j0 + 1, 1)
    consume(j0 + 1, 1)

  plsc.subcore_barrier()

  pltpu.sync_copy(acc_sh.at[pl.ds(base, ROWS_PER_SUBCORE)],
                  part_hbm.at[c, pl.ds(base, ROWS_PER_SUBCORE)])

  @pl.when(c == 0)
  def _():
    pltpu.sync_copy(hist_v, deg_hbm.at[s])


def _combine_body(part_ref, d_ref, o_ref):
  deg = jnp.sum(d_ref[...], axis=0)[:N_NODES]
  deg = jnp.maximum(deg, 1.0).reshape(N_NODES, 1)
  halves = jnp.concatenate(
      [part_ref[0, :N_NODES, :], part_ref[1, :N_NODES, :]], axis=1)
  o_ref[...] = halves / deg


@jax.jit
def kernel(user_emb, edge_index):
  src = edge_index[0].astype(jnp.int32)
  dst = edge_index[1].astype(jnp.int32)
  pad = E_PAD - N_EDGES
  src = jnp.concatenate([src, jnp.zeros((pad,), jnp.int32)])
  dst = jnp.concatenate([dst, jnp.full((pad,), DUMMY_ROW, jnp.int32)])
  src_r = src.reshape(NUM_SUBCORES, CHUNKS_PER_TILE, CHUNK)
  dst_r = dst.reshape(NUM_SUBCORES, CHUNKS_PER_TILE, CHUNK)

  emb_t = user_emb.reshape(N_NODES, NUM_CORES, D_HALF).transpose(1, 0, 2)

  zhalf = jnp.zeros((CHUNK, D_HALF), jnp.float32)

  mesh = plsc.VectorSubcoreMesh(core_axis_name="c", subcore_axis_name="s")
  sc = pl.kernel(
      _sc_body,
      out_type=[
          jax.ShapeDtypeStruct((NUM_CORES, ACC_ROWS, D_HALF), jnp.float32),
          jax.ShapeDtypeStruct((NUM_SUBCORES, ACC_ROWS), jnp.float32),
      ],
      mesh=mesh,
      compiler_params=pltpu.CompilerParams(use_tc_tiling_on_sc=False,
                                           needs_layout_passes=False),
      scratch_types=[
          pltpu.VMEM((CHUNKS_PER_TILE, CHUNK), jnp.int32),
          pltpu.VMEM((CHUNKS_PER_TILE, CHUNK), jnp.int32),
          pltpu.VMEM((CHUNK, D_HALF), jnp.float32),
          pltpu.VMEM((CHUNK, D_HALF), jnp.float32),
          pltpu.VMEM((CHUNK, D_HALF), jnp.float32),
          pltpu.VMEM((ACC_ROWS,), jnp.float32),
          pltpu.VMEM_SHARED((ACC_ROWS, D_HALF), jnp.float32),
          pltpu.SemaphoreType.DMA,
          pltpu.SemaphoreType.DMA,
      ],
  )
  part, deg = sc(emb_t, src_r, dst_r, zhalf)

  out = pl.pallas_call(
      _combine_body,
      out_shape=jax.ShapeDtypeStruct((N_NODES, D_FEAT), jnp.float32),
  )(part, deg)
  return out


# speedup vs baseline: 1.2173x; 1.0088x over previous
import jax
import jax.numpy as jnp
from jax import lax
from jax.experimental import pallas as pl
from jax.experimental.pallas import tpu as pltpu
from jax.experimental.pallas import tpu_sc as plsc

N_NODES = 10000
N_EDGES = 320000
D_FEAT = 128

NUM_CORES = 2
NUM_SUBCORES = 16
NUM_LANES = 16
D_HALF = D_FEAT // NUM_CORES

CHUNK = 128
CHUNKS_PER_TILE = 2 * -(-N_EDGES // (NUM_SUBCORES * CHUNK * 2))
EDGES_PER_TILE = CHUNKS_PER_TILE * CHUNK
E_PAD = EDGES_PER_TILE * NUM_SUBCORES

ACC_ROWS = 10112
ROWS_PER_SUBCORE = ACC_ROWS // NUM_SUBCORES
DUMMY_ROW = N_NODES


def _sc_body(emb_hbm, src_hbm, dst_hbm, zeros_hbm,
             part_hbm, deg_hbm,
             src_v, dst_v, rows0_v, rows1_v, zb_v, hist_v, acc_sh,
             gsem0, gsem1):
  c = lax.axis_index("c")
  s = lax.axis_index("s")
  rows = [rows0_v, rows1_v]
  gsem = [gsem0, gsem1]

  pltpu.sync_copy(zeros_hbm, zb_v)

  @pl.loop(0, ACC_ROWS // NUM_LANES)
  def _(i):
    hist_v[pl.ds(i * NUM_LANES, NUM_LANES)] = jnp.zeros((NUM_LANES,),
                                                        jnp.float32)

  base = s * ROWS_PER_SUBCORE
  for k in range(ROWS_PER_SUBCORE // CHUNK):
    pltpu.sync_copy(zb_v, acc_sh.at[pl.ds(base + k * CHUNK, CHUNK)])
  rem = ROWS_PER_SUBCORE % CHUNK
  if rem:
    off = base + (ROWS_PER_SUBCORE // CHUNK) * CHUNK
    pltpu.sync_copy(zb_v.at[pl.ds(0, rem)], acc_sh.at[pl.ds(off, rem)])

  plsc.subcore_barrier()

  pltpu.sync_copy(src_hbm.at[s], src_v)
  pltpu.sync_copy(dst_hbm.at[s], dst_v)

  emb_half = emb_hbm.at[c]

  def gather(j, b):
    pltpu.async_copy(emb_half.at[src_v.at[j]], rows[b], gsem[b])

  def wait_gather(j, b):
    pltpu.make_async_copy(emb_half.at[src_v.at[j]], rows[b],
                          gsem[b]).wait()

  def consume(j, b):
    @pl.when(c == (b % 2))
    def _():
      ones = jnp.ones((NUM_LANES,), jnp.float32)
      for k in range(CHUNK // NUM_LANES):
        idx = dst_v[j, pl.ds(k * NUM_LANES, NUM_LANES)]
        plsc.addupdate_scatter(hist_v, [idx], ones)

    pltpu.sync_copy(rows[b], acc_sh.at[dst_v.at[j]], add=True)

  gather(0, 0)

  @pl.loop(0, CHUNKS_PER_TILE // 2)
  def _(i):
    j0 = 2 * i
    gather(j0 + 1, 1)
    wait_gather(j0, 0)
    consume(j0, 0)

    @pl.when(j0 + 2 < CHUNKS_PER_TILE)
    def _():
      gather(j0 + 2, 0)

    wait_gather(j0 + 1, 1)
    consume(j0 + 1, 1)

  plsc.subcore_barrier()

  pltpu.sync_copy(acc_sh.at[pl.ds(base, ROWS_PER_SUBCORE)],
                  part_hbm.at[c, pl.ds(base, ROWS_PER_SUBCORE)])
  pltpu.sync_copy(hist_v, deg_hbm.at[c, s])


def _combine_body(part_ref, d_ref, o_ref):
  deg = jnp.sum(d_ref[...].reshape(NUM_CORES * NUM_SUBCORES, ACC_ROWS),
                axis=0)[:N_NODES]
  deg = jnp.maximum(deg, 1.0).reshape(N_NODES, 1)
  halves = jnp.concatenate(
      [part_ref[0, :N_NODES, :], part_ref[1, :N_NODES, :]], axis=1)
  o_ref[...] = halves / deg


@jax.jit
def kernel(user_emb, edge_index):
  src = edge_index[0].astype(jnp.int32).reshape(N_EDGES // CHUNK, CHUNK)
  dst = edge_index[1].astype(jnp.int32).reshape(N_EDGES // CHUNK, CHUNK)
  pad_rows = NUM_SUBCORES * CHUNKS_PER_TILE - N_EDGES // CHUNK
  src_r = jnp.pad(src, ((0, pad_rows), (0, 0))).reshape(
      NUM_SUBCORES, CHUNKS_PER_TILE, CHUNK)
  dst_r = jnp.pad(dst, ((0, pad_rows), (0, 0)),
                  constant_values=DUMMY_ROW).reshape(
      NUM_SUBCORES, CHUNKS_PER_TILE, CHUNK)

  emb_t = user_emb.reshape(N_NODES, NUM_CORES, D_HALF).transpose(1, 0, 2)

  zhalf = jnp.zeros((CHUNK, D_HALF), jnp.float32)

  mesh = plsc.VectorSubcoreMesh(core_axis_name="c", subcore_axis_name="s")
  sc = pl.kernel(
      _sc_body,
      out_type=[
          jax.ShapeDtypeStruct((NUM_CORES, ACC_ROWS, D_HALF), jnp.float32),
          jax.ShapeDtypeStruct((NUM_CORES, NUM_SUBCORES, ACC_ROWS),
                               jnp.float32),
      ],
      mesh=mesh,
      compiler_params=pltpu.CompilerParams(use_tc_tiling_on_sc=False,
                                           needs_layout_passes=False),
      scratch_types=[
          pltpu.VMEM((CHUNKS_PER_TILE, CHUNK), jnp.int32),
          pltpu.VMEM((CHUNKS_PER_TILE, CHUNK), jnp.int32),
          pltpu.VMEM((CHUNK, D_HALF), jnp.float32),
          pltpu.VMEM((CHUNK, D_HALF), jnp.float32),
          pltpu.VMEM((CHUNK, D_HALF), jnp.float32),
          pltpu.VMEM((ACC_ROWS,), jnp.float32),
          pltpu.VMEM_SHARED((ACC_ROWS, D_HALF), jnp.float32),
          pltpu.SemaphoreType.DMA,
          pltpu.SemaphoreType.DMA,
      ],
  )
  part, deg = sc(emb_t, src_r, dst_r, zhalf)

  out = pl.pallas_call(
      _combine_body,
      out_shape=jax.ShapeDtypeStruct((N_NODES, D_FEAT), jnp.float32),
  )(part, deg)
  return out


# speedup vs baseline: 1.8110x; 1.4878x over previous
import jax
import jax.numpy as jnp
from jax import lax
from jax.experimental import pallas as pl
from jax.experimental.pallas import tpu as pltpu
from jax.experimental.pallas import tpu_sc as plsc

N_NODES = 10000
N_EDGES = 320000
D_FEAT = 128

NUM_CORES = 2
NUM_SUBCORES = 16
NUM_LANES = 16
D_HALF = D_FEAT // NUM_CORES

CHUNK = 128
TOTAL_CHUNKS = N_EDGES // CHUNK
MAIN_CHUNKS = TOTAL_CHUNKS // NUM_SUBCORES
LEFT = TOTAL_CHUNKS - MAIN_CHUNKS * NUM_SUBCORES
CAP_CHUNKS = MAIN_CHUNKS + 1
MAIN_EDGES = MAIN_CHUNKS * CHUNK

ACC_ROWS = 10112
ROWS_PER_SUBCORE = ACC_ROWS // NUM_SUBCORES


def _sc_body(emb_hbm, edge_hbm, zeros_hbm,
             part_hbm, deg_hbm,
             src_v, dst_v, rows0_v, rows1_v, zb_v, hist_v, acc_sh,
             gsem0, gsem1):
  c = lax.axis_index("c")
  s = lax.axis_index("s")
  rows = [rows0_v, rows1_v]
  gsem = [gsem0, gsem1]

  pltpu.sync_copy(zeros_hbm, zb_v)

  @pl.loop(0, ACC_ROWS // NUM_LANES)
  def _(i):
    hist_v[pl.ds(i * NUM_LANES, NUM_LANES)] = jnp.zeros((NUM_LANES,),
                                                        jnp.float32)

  base = s * ROWS_PER_SUBCORE
  for k in range(ROWS_PER_SUBCORE // CHUNK):
    pltpu.sync_copy(zb_v, acc_sh.at[pl.ds(base + k * CHUNK, CHUNK)])
  rem = ROWS_PER_SUBCORE % CHUNK
  if rem:
    off = base + (ROWS_PER_SUBCORE // CHUNK) * CHUNK
    pltpu.sync_copy(zb_v.at[pl.ds(0, rem)], acc_sh.at[pl.ds(off, rem)])

  plsc.subcore_barrier()

  pltpu.sync_copy(edge_hbm.at[0, pl.ds(s * MAIN_EDGES, MAIN_EDGES)],
                  src_v.at[pl.ds(0, MAIN_EDGES)])
  pltpu.sync_copy(edge_hbm.at[1, pl.ds(s * MAIN_EDGES, MAIN_EDGES)],
                  dst_v.at[pl.ds(0, MAIN_EDGES)])

  @pl.when(s < LEFT)
  def _():
    off = MAIN_EDGES * NUM_SUBCORES + s * CHUNK
    pltpu.sync_copy(edge_hbm.at[0, pl.ds(off, CHUNK)],
                    src_v.at[pl.ds(MAIN_EDGES, CHUNK)])
    pltpu.sync_copy(edge_hbm.at[1, pl.ds(off, CHUNK)],
                    dst_v.at[pl.ds(MAIN_EDGES, CHUNK)])

  n_chunks = jnp.where(s < LEFT, CAP_CHUNKS, MAIN_CHUNKS)

  emb_half = emb_hbm.at[c]

  def sidx(j):
    return src_v.at[pl.ds(j * CHUNK, CHUNK)]

  def didx(j):
    return dst_v.at[pl.ds(j * CHUNK, CHUNK)]

  def gather(j, b):
    pltpu.async_copy(emb_half.at[sidx(j)], rows[b], gsem[b])

  def wait_gather(j, b):
    pltpu.make_async_copy(emb_half.at[sidx(j)], rows[b], gsem[b]).wait()

  def consume(j, b):
    @pl.when(c == (b % 2))
    def _():
      ones = jnp.ones((NUM_LANES,), jnp.float32)
      for k in range(CHUNK // NUM_LANES):
        idx = dst_v[pl.ds(j * CHUNK + k * NUM_LANES, NUM_LANES)]
        plsc.addupdate_scatter(hist_v, [idx], ones)

    pltpu.sync_copy(rows[b], acc_sh.at[didx(j)], add=True)

  gather(0, 0)

  @pl.loop(0, MAIN_CHUNKS // 2)
  def _(i):
    j0 = 2 * i
    gather(j0 + 1, 1)
    wait_gather(j0, 0)
    consume(j0, 0)

    @pl.when(j0 + 2 < n_chunks)
    def _():
      gather(j0 + 2, 0)

    wait_gather(j0 + 1, 1)
    consume(j0 + 1, 1)

  @pl.when(s < LEFT)
  def _():
    wait_gather(MAIN_CHUNKS, 0)
    consume(MAIN_CHUNKS, 0)

  plsc.subcore_barrier()

  pltpu.sync_copy(acc_sh.at[pl.ds(base, ROWS_PER_SUBCORE)],
                  part_hbm.at[c, pl.ds(base, ROWS_PER_SUBCORE)])
  pltpu.sync_copy(hist_v, deg_hbm.at[c, s])


def _combine_body(part_ref, d_ref, o_ref):
  deg = jnp.sum(d_ref[...].reshape(NUM_CORES * NUM_SUBCORES, ACC_ROWS),
                axis=0)[:N_NODES]
  deg = jnp.maximum(deg, 1.0).reshape(N_NODES, 1)
  halves = jnp.concatenate(
      [part_ref[0, :N_NODES, :], part_ref[1, :N_NODES, :]], axis=1)
  o_ref[...] = halves / deg


@jax.jit
def kernel(user_emb, edge_index):
  edge_i32 = edge_index.astype(jnp.int32)

  emb_t = user_emb.reshape(N_NODES, NUM_CORES, D_HALF).transpose(1, 0, 2)

  zhalf = jnp.zeros((CHUNK, D_HALF), jnp.float32)

  mesh = plsc.VectorSubcoreMesh(core_axis_name="c", subcore_axis_name="s")
  sc = pl.kernel(
      _sc_body,
      out_type=[
          jax.ShapeDtypeStruct((NUM_CORES, ACC_ROWS, D_HALF), jnp.float32),
          jax.ShapeDtypeStruct((NUM_CORES, NUM_SUBCORES, ACC_ROWS),
                               jnp.float32),
      ],
      mesh=mesh,
      compiler_params=pltpu.CompilerParams(use_tc_tiling_on_sc=False,
                                           needs_layout_passes=False),
      scratch_types=[
          pltpu.VMEM((CAP_CHUNKS * CHUNK,), jnp.int32),
          pltpu.VMEM((CAP_CHUNKS * CHUNK,), jnp.int32),
          pltpu.VMEM((CHUNK, D_HALF), jnp.float32),
          pltpu.VMEM((CHUNK, D_HALF), jnp.float32),
          pltpu.VMEM((CHUNK, D_HALF), jnp.float32),
          pltpu.VMEM((ACC_ROWS,), jnp.float32),
          pltpu.VMEM_SHARED((ACC_ROWS, D_HALF), jnp.float32),
          pltpu.SemaphoreType.DMA,
          pltpu.SemaphoreType.DMA,
      ],
  )
  part, deg = sc(emb_t, edge_i32, zhalf)

  out = pl.pallas_call(
      _combine_body,
      out_shape=jax.ShapeDtypeStruct((N_NODES, D_FEAT), jnp.float32),
  )(part, deg)
  return out


# speedup vs baseline: 2.0616x; 1.1384x over previous
import jax
import jax.numpy as jnp
from jax import lax
from jax.experimental import pallas as pl
from jax.experimental.pallas import tpu as pltpu
from jax.experimental.pallas import tpu_sc as plsc

N_NODES = 10000
N_EDGES = 320000
D_FEAT = 128

NUM_CORES = 2
NUM_SUBCORES = 16
NUM_LANES = 16
D_HALF = D_FEAT // NUM_CORES

CHUNK = 128
TOTAL_CHUNKS = N_EDGES // CHUNK
MAIN_CHUNKS = TOTAL_CHUNKS // NUM_SUBCORES
LEFT = TOTAL_CHUNKS - MAIN_CHUNKS * NUM_SUBCORES
CAP_CHUNKS = MAIN_CHUNKS + 1
MAIN_EDGES = MAIN_CHUNKS * CHUNK

ACC_ROWS = 10112
ROWS_PER_SUBCORE = ACC_ROWS // NUM_SUBCORES


def _sc_body(emb_hbm, edge_hbm, zeros_hbm,
             part_hbm, deg_hbm,
             src_v, dst_v, rows0_v, rows1_v, rows2_v, zb_v, hist_v, acc_sh,
             gsem0, gsem1, gsem2, ssem0, ssem1, ssem2):
  c = lax.axis_index("c")
  s = lax.axis_index("s")
  rows = [rows0_v, rows1_v, rows2_v]
  gsem = [gsem0, gsem1, gsem2]
  ssem = [ssem0, ssem1, ssem2]

  pltpu.sync_copy(zeros_hbm, zb_v)

  @pl.loop(0, ACC_ROWS // NUM_LANES)
  def _(i):
    hist_v[pl.ds(i * NUM_LANES, NUM_LANES)] = jnp.zeros((NUM_LANES,),
                                                        jnp.float32)

  base = s * ROWS_PER_SUBCORE
  for k in range(ROWS_PER_SUBCORE // CHUNK):
    pltpu.sync_copy(zb_v, acc_sh.at[pl.ds(base + k * CHUNK, CHUNK)])
  rem = ROWS_PER_SUBCORE % CHUNK
  if rem:
    off = base + (ROWS_PER_SUBCORE // CHUNK) * CHUNK
    pltpu.sync_copy(zb_v.at[pl.ds(0, rem)], acc_sh.at[pl.ds(off, rem)])

  plsc.subcore_barrier()

  pltpu.sync_copy(edge_hbm.at[0, pl.ds(s * MAIN_EDGES, MAIN_EDGES)],
                  src_v.at[pl.ds(0, MAIN_EDGES)])
  pltpu.sync_copy(edge_hbm.at[1, pl.ds(s * MAIN_EDGES, MAIN_EDGES)],
                  dst_v.at[pl.ds(0, MAIN_EDGES)])

  @pl.when(s < LEFT)
  def _():
    off = MAIN_EDGES * NUM_SUBCORES + s * CHUNK
    pltpu.sync_copy(edge_hbm.at[0, pl.ds(off, CHUNK)],
                    src_v.at[pl.ds(MAIN_EDGES, CHUNK)])
    pltpu.sync_copy(edge_hbm.at[1, pl.ds(off, CHUNK)],
                    dst_v.at[pl.ds(MAIN_EDGES, CHUNK)])

  n_chunks = jnp.where(s < LEFT, CAP_CHUNKS, MAIN_CHUNKS)

  emb_half = emb_hbm.at[c]

  def sidx(j):
    return src_v.at[pl.ds(j * CHUNK, CHUNK)]

  def didx(j):
    return dst_v.at[pl.ds(j * CHUNK, CHUNK)]

  def gather(j, b):
    pltpu.async_copy(emb_half.at[sidx(j)], rows[b], gsem[b])

  def wait_gather(j, b):
    pltpu.make_async_copy(emb_half.at[sidx(j)], rows[b], gsem[b]).wait()

  def scatter(j, b):
    pltpu.async_copy(rows[b], acc_sh.at[didx(j)], ssem[b], add=True)

  def wait_scatter(b):
    pltpu.make_async_copy(rows[b], acc_sh.at[didx(0)], ssem[b]).wait()

  def hist(j, u):
    @pl.when(c == (u % 2))
    def _():
      ones = jnp.ones((NUM_LANES,), jnp.float32)
      for k in range(CHUNK // NUM_LANES):
        idx = dst_v[pl.ds(j * CHUNK + k * NUM_LANES, NUM_LANES)]
        plsc.addupdate_scatter(hist_v, [idx], ones)

  gather(0, 0)
  gather(1, 1)

  @pl.loop(0, MAIN_CHUNKS // 3)
  def _(i):
    for u in range(3):
      j = 3 * i + u
      b = u
      nb = (u + 2) % 3
      wait_gather(j, b)
      scatter(j, b)
      hist(j, u)

      @pl.when(j + 2 < n_chunks)
      def _():
        @pl.when(j >= 1)
        def _():
          wait_scatter(nb)

        gather(j + 2, nb)

  @pl.when(s < LEFT)
  def _():
    wait_gather(MAIN_CHUNKS, MAIN_CHUNKS % 3)
    scatter(MAIN_CHUNKS, MAIN_CHUNKS % 3)
    hist(MAIN_CHUNKS, MAIN_CHUNKS % 3)

  for b in range(3):
    wait_scatter(b)

  plsc.subcore_barrier()

  pltpu.sync_copy(acc_sh.at[pl.ds(base, ROWS_PER_SUBCORE)],
                  part_hbm.at[c, pl.ds(base, ROWS_PER_SUBCORE)])
  pltpu.sync_copy(hist_v, deg_hbm.at[c, s])


def _combine_body(part_ref, d_ref, o_ref):
  deg = jnp.sum(d_ref[...].reshape(NUM_CORES * NUM_SUBCORES, ACC_ROWS),
                axis=0)[:N_NODES]
  deg = jnp.maximum(deg, 1.0).reshape(N_NODES, 1)
  halves = jnp.concatenate(
      [part_ref[0, :N_NODES, :], part_ref[1, :N_NODES, :]], axis=1)
  o_ref[...] = halves / deg


@jax.jit
def kernel(user_emb, edge_index):
  edge_i32 = edge_index.astype(jnp.int32)

  emb_t = user_emb.reshape(N_NODES, NUM_CORES, D_HALF).transpose(1, 0, 2)

  zhalf = jnp.zeros((CHUNK, D_HALF), jnp.float32)

  mesh = plsc.VectorSubcoreMesh(core_axis_name="c", subcore_axis_name="s")
  sc = pl.kernel(
      _sc_body,
      out_type=[
          jax.ShapeDtypeStruct((NUM_CORES, ACC_ROWS, D_HALF), jnp.float32),
          jax.ShapeDtypeStruct((NUM_CORES, NUM_SUBCORES, ACC_ROWS),
                               jnp.float32),
      ],
      mesh=mesh,
      compiler_params=pltpu.CompilerParams(use_tc_tiling_on_sc=False,
                                           needs_layout_passes=False),
      scratch_types=[
          pltpu.VMEM((CAP_CHUNKS * CHUNK,), jnp.int32),
          pltpu.VMEM((CAP_CHUNKS * CHUNK,), jnp.int32),
          pltpu.VMEM((CHUNK, D_HALF), jnp.float32),
          pltpu.VMEM((CHUNK, D_HALF), jnp.float32),
          pltpu.VMEM((CHUNK, D_HALF), jnp.float32),
          pltpu.VMEM((CHUNK, D_HALF), jnp.float32),
          pltpu.VMEM((ACC_ROWS,), jnp.float32),
          pltpu.VMEM_SHARED((ACC_ROWS, D_HALF), jnp.float32),
      ] + [pltpu.SemaphoreType.DMA] * 6,
  )
  part, deg = sc(emb_t, edge_i32, zhalf)

  out = pl.pallas_call(
      _combine_body,
      out_shape=jax.ShapeDtypeStruct((N_NODES, D_FEAT), jnp.float32),
  )(part, deg)
  return out
